# baseline (device time: 96936 ns/iter reference)
import jax
import jax.numpy as jnp
from jax import lax
from jax.experimental import pallas as pl
from jax.experimental.pallas import tpu as pltpu

Z = 4
M = 2048
Q = M // 4
C = 4
QC = Q // C


def kernel(x, pi):
    def body(
        x_ref,
        pi_ref,
        out_ref,
        p1_send,
        p1_recv,
        p2_send,
        p2_recv,
        p3a_send,
        p3a_recv,
        p3b_send,
        p3b_recv,
    ):
        my_x = lax.axis_index("x")
        my_y = lax.axis_index("y")
        my_z = lax.axis_index("z")
        r = (Z - pi_ref[0]) % Z
        dst_z = (my_z - r) % Z

        ypar = my_y % 2
        q = 2 * my_x + ypar
        y_partner = my_y + 1 - 2 * ypar
        x_partner = 1 - my_x

        def copy(rows_start, nbr, send_sems, recv_sems, c, src=None):
            src = x_ref if src is None else src
            return pltpu.make_async_remote_copy(
                src_ref=src.at[:, pl.ds(rows_start, QC), :],
                dst_ref=out_ref.at[:, pl.ds(rows_start, QC), :],
                send_sem=send_sems.at[c],
                recv_sem=recv_sems.at[c],
                device_id=nbr,
                device_id_type=pl.DeviceIdType.MESH,
            )

        col = (my_x, my_y, dst_z)
        ynbr = (my_x, y_partner, my_z)
        xnbr = (x_partner, my_y, my_z)
        dnbr = (x_partner, y_partner, my_z)

        p1 = [copy(q * Q + c * QC, col, p1_send, p1_recv, c) for c in range(C)]
        for d in p1:
            d.start()

        py, px, pd = [], [], []
        for c in range(C):
            p1[c].wait()
            rows = q * Q + c * QC
            dy = copy(rows, ynbr, p2_send, p2_recv, c, src=out_ref)
            dy.start()
            py.append(dy)
            dx = copy(rows, xnbr, p3a_send, p3a_recv, c, src=out_ref)
            dx.start()
            px.append(dx)
            dd = copy(rows, dnbr, p3b_send, p3b_recv, c, src=out_ref)
            dd.start()
            pd.append(dd)

        for d in py:
            d.wait()
        for d in px:
            d.wait()
        for d in pd:
            d.wait()

    return pl.pallas_call(
        body,
        out_shape=jax.ShapeDtypeStruct(x.shape, x.dtype),
        in_specs=[
            pl.BlockSpec(memory_space=pltpu.VMEM),
            pl.BlockSpec(memory_space=pltpu.SMEM),
        ],
        out_specs=pl.BlockSpec(memory_space=pltpu.VMEM),
        scratch_shapes=[pltpu.SemaphoreType.DMA((C,))] * 8,
    )(x, pi)
